# baseline (speedup 1.0000x reference)
import functools

import jax
import jax.numpy as jnp
from jax import lax
from jax.experimental import pallas as pl
from jax.experimental.pallas import tpu as pltpu
from jax.experimental.pallas import tpu_sc as plsc

D = 32
BATCH = 4096
SEQ = 200
NC = 2
NS = 16
NW = NC * NS
BT = BATCH // NW
ST = SEQ // 8
DT = D // 8
NB = 4

_mesh = plsc.VectorSubcoreMesh(core_axis_name="c", subcore_axis_name="s")


@functools.partial(
    pl.kernel,
    out_type=(
        jax.ShapeDtypeStruct((SEQ, DT, NW, 8, BT), jnp.float32),
        jax.ShapeDtypeStruct((DT, NW, 8, BT), jnp.float32),
    ),
    mesh=_mesh,
    compiler_params=pltpu.CompilerParams(
        use_tc_tiling_on_sc=False, needs_layout_passes=False),
    scratch_types=[
        pltpu.VMEM((ST, 8, BT), jnp.int32),
        pltpu.VMEM((NB, BT, D), jnp.float32),
        pltpu.VMEM((NB, DT, 8, BT), jnp.float32),
        pltpu.VMEM((DT, 8, BT), jnp.float32),
        pltpu.SemaphoreType.DMA,
        pltpu.SemaphoreType.DMA,
        pltpu.SemaphoreType.DMA,
        pltpu.SemaphoreType.DMA,
        pltpu.SemaphoreType.DMA,
        pltpu.SemaphoreType.DMA,
        pltpu.SemaphoreType.DMA,
        pltpu.SemaphoreType.DMA,
    ],
)
def _embed_pool(ids_hbm, table_hbm, x_hbm, mean_hbm,
                idx_v, buf_v, tbuf_v, macc_v,
                g0, g1, g2, g3, w0, w1, w2, w3):
    gsems = (g0, g1, g2, g3)
    wsems = (w0, w1, w2, w3)
    wid = lax.axis_index("s") * NC + lax.axis_index("c")
    inv = jnp.float32(1.0 / SEQ)
    zero16 = jnp.zeros((16,), jnp.float32)

    pltpu.sync_copy(ids_hbm.at[:, wid], idx_v)

    def zstep(t, _):
        macc_v[t // 64, (t // 8) % 8, pl.ds((t % 8) * 16, 16)] = zero16
        return 0
    lax.fori_loop(0, DT * 8 * 8, zstep, 0)

    def start_gather(s, b):
        pltpu.async_copy(table_hbm.at[idx_v.at[s // 8, s % 8]],
                         buf_v.at[b], gsems[b])

    for b in range(NB):
        start_gather(b, b)

    lane = lax.iota(jnp.int32, 16)

    @pl.loop(0, SEQ, step=NB)
    def _round(s0):
        for b in range(NB):
            s = s0 + b
            buf = buf_v.at[b]
            tbuf = tbuf_v.at[b]
            pltpu.make_async_copy(
                table_hbm.at[pl.ds(0, BT)], buf, gsems[b]).wait()

            @pl.when(s >= NB)
            def _():
                pltpu.make_async_copy(
                    x_hbm.at[0, :, 0], tbuf, wsems[b]).wait()

            def dstep(d, _, buf=buf, tbuf=tbuf):
                dt = d // 8
                di = d % 8
                dvec = jnp.full((16,), d, jnp.int32)
                for g in range(8):
                    v = plsc.load_gather(buf, [g * 16 + lane, dvec])
                    tbuf[dt, di, pl.ds(g * 16, 16)] = v
                    plsc.addupdate(macc_v.at[dt, di, pl.ds(g * 16, 16)], v)
                return 0
            lax.fori_loop(0, D, dstep, 0)

            pltpu.async_copy(tbuf, x_hbm.at[s, :, wid], wsems[b])

            @pl.when(s + NB < SEQ)
            def _():
                start_gather(s + NB, b)

    for b in range(NB):
        pltpu.make_async_copy(x_hbm.at[0, :, 0], tbuf_v.at[b], wsems[b]).wait()

    def mstep(t, _):
        dt = t // 64
        di = (t // 8) % 8
        o = (t % 8) * 16
        macc_v[dt, di, pl.ds(o, 16)] = macc_v[dt, di, pl.ds(o, 16)] * inv
        return 0
    lax.fori_loop(0, DT * 8 * 8, mstep, 0)
    pltpu.sync_copy(macc_v, mean_hbm.at[:, wid])


def kernel(input_ids, embedding_weight):
    ids5 = input_ids.T.reshape(ST, 8, NW, BT).transpose(0, 2, 1, 3)
    x5, m4 = _embed_pool(ids5, embedding_weight)
    x = x5.transpose(2, 4, 0, 1, 3).reshape(BATCH, SEQ, D)
    mean = m4.transpose(1, 3, 0, 2).reshape(BATCH, D)
    return x, mean


# speedup vs baseline: 1.5592x; 1.5592x over previous
import functools

import jax
import jax.numpy as jnp
from jax import lax
from jax.experimental import pallas as pl
from jax.experimental.pallas import tpu as pltpu
from jax.experimental.pallas import tpu_sc as plsc

D = 32
BATCH = 4096
SEQ = 200
NC = 2
NS = 16
NW = NC * NS
BT = BATCH // NW
ST = SEQ // 8
DT = D // 8
NB = 4
JB = BT // 16
DB = D // 16
NBLK = JB * DB

_mesh = plsc.VectorSubcoreMesh(core_axis_name="c", subcore_axis_name="s")


@functools.partial(
    pl.kernel,
    out_type=(
        jax.ShapeDtypeStruct((SEQ, DT, NW, 8, BT), jnp.float32),
        jax.ShapeDtypeStruct((DT, NW, 8, BT), jnp.float32),
    ),
    mesh=_mesh,
    compiler_params=pltpu.CompilerParams(
        use_tc_tiling_on_sc=False, needs_layout_passes=False),
    scratch_types=[
        pltpu.VMEM((ST, 8, BT), jnp.int32),
        pltpu.VMEM((NB, BT, D), jnp.float32),
        pltpu.VMEM((NB, DT, 8, BT), jnp.float32),
        pltpu.VMEM((NBLK * 16, 16), jnp.float32),
        pltpu.VMEM((DT, 8, BT), jnp.float32),
        pltpu.SemaphoreType.DMA,
        pltpu.SemaphoreType.DMA,
        pltpu.SemaphoreType.DMA,
        pltpu.SemaphoreType.DMA,
        pltpu.SemaphoreType.DMA,
        pltpu.SemaphoreType.DMA,
        pltpu.SemaphoreType.DMA,
        pltpu.SemaphoreType.DMA,
    ],
)
def _embed_pool(ids_hbm, table_hbm, x_hbm, mean_hbm,
                idx_v, buf_v, tbuf_v, macc_v, mtb_v,
                g0, g1, g2, g3, w0, w1, w2, w3):
    gsems = (g0, g1, g2, g3)
    wsems = (w0, w1, w2, w3)
    wid = lax.axis_index("s") * NC + lax.axis_index("c")
    inv = jnp.float32(1.0 / SEQ)
    zero16 = jnp.zeros((16,), jnp.float32)
    lane = lax.iota(jnp.int32, 16)

    pltpu.sync_copy(ids_hbm.at[:, wid], idx_v)

    def zstep(r, _):
        macc_v[r, :] = zero16
        return 0
    lax.fori_loop(0, NBLK * 16, zstep, 0)

    def start_gather(s, b):
        pltpu.async_copy(table_hbm.at[idx_v.at[s // 8, s % 8]],
                         buf_v.at[b], gsems[b])

    for b in range(NB):
        start_gather(b, b)

    @pl.loop(0, SEQ, step=NB)
    def _round(s0):
        for b in range(NB):
            s = s0 + b
            buf = buf_v.at[b]
            tbuf = tbuf_v.at[b]
            pltpu.make_async_copy(
                table_hbm.at[pl.ds(0, BT)], buf, gsems[b]).wait()

            @pl.when(s >= NB)
            def _():
                pltpu.make_async_copy(
                    x_hbm.at[0, :, 0], tbuf, wsems[b]).wait()

            def tstep(t, _, buf=buf, tbuf=tbuf):
                rot = (lane + t) & 15
                for jb in range(JB):
                    for db in range(DB):
                        j0 = jb * 16
                        dvec = db * 16 + rot
                        v = plsc.load_gather(buf, [j0 + lane, dvec])
                        plsc.store_scatter(
                            tbuf, [dvec >> 3, dvec & 7, j0 + lane], v)
                        plsc.addupdate(
                            macc_v.at[(jb * DB + db) * 16 + t, :], v)
                return 0
            lax.fori_loop(0, 16, tstep, 0)

            pltpu.async_copy(tbuf, x_hbm.at[s, :, wid], wsems[b])

            @pl.when(s + NB < SEQ)
            def _():
                start_gather(s + NB, b)

    for b in range(NB):
        pltpu.make_async_copy(x_hbm.at[0, :, 0], tbuf_v.at[b], wsems[b]).wait()

    def mstep(t, _):
        rot = (lane + t) & 15
        for jb in range(JB):
            for db in range(DB):
                j0 = jb * 16
                dvec = db * 16 + rot
                v = macc_v[(jb * DB + db) * 16 + t, :] * inv
                plsc.store_scatter(
                    mtb_v, [dvec >> 3, dvec & 7, j0 + lane], v)
        return 0
    lax.fori_loop(0, 16, mstep, 0)
    pltpu.sync_copy(mtb_v, mean_hbm.at[:, wid])


def kernel(input_ids, embedding_weight):
    ids5 = input_ids.T.reshape(ST, 8, NW, BT).transpose(0, 2, 1, 3)
    x5, m4 = _embed_pool(ids5, embedding_weight)
    x = x5.transpose(2, 4, 0, 1, 3).reshape(BATCH, SEQ, D)
    mean = m4.transpose(1, 3, 0, 2).reshape(BATCH, D)
    return x, mean
